# Initial kernel scaffold; baseline (speedup 1.0000x reference)
import jax
import jax.numpy as jnp
from jax.experimental import pallas as pl


def kernel(scores, order):
    raise NotImplementedError("write your pallas kernel here")


# speedup vs baseline: 79.6422x; 79.6422x over previous
import jax
import jax.numpy as jnp
from jax import lax
from jax.experimental import pallas as pl
from jax.experimental.pallas import tpu as pltpu, tpu_sc as plsc

B, N = 1024, 200
L = 16
NFULL = N // L
TAIL_OFF = N - L
TAIL_LO = L - (N - NFULL * L)

_LN2 = 0.6931471805599453
_SQRT2 = 1.4142135623730951


def _vlog(x):
    bits = plsc.bitcast(x, jnp.int32)
    k = lax.shift_right_logical(bits, 23) - 127
    fbits = lax.bitwise_or(lax.bitwise_and(bits, 0x007FFFFF), 0x3F800000)
    f = plsc.bitcast(fbits, jnp.float32)
    big = f > _SQRT2
    f = jnp.where(big, f * 0.5, f)
    kf = k.astype(jnp.float32) + jnp.where(big, 1.0, 0.0)
    t = (f - 1.0) / (f + 1.0)
    t2 = t * t
    p = 0.14285715 + t2 * 0.11111111
    p = 0.2 + t2 * p
    p = 0.33333334 + t2 * p
    return kf * _LN2 + (t + t * t2 * p) * 2.0


def _gather(v, idx):
    return v.at[idx].get(mode="promise_in_bounds")


def _splat_max(v):
    lanes = lax.iota(jnp.int32, L)
    for sh in (1, 2, 4, 8):
        v = jnp.maximum(v, _gather(v, lax.bitwise_xor(lanes, sh)))
    return v


def _splat_last(v):
    return _gather(v, jnp.full((L,), L - 1, jnp.int32))


def _row_loss(scores_v, order_v, esort_v, i, lanes):
    tail_valid = lanes >= TAIL_LO

    m_vec = scores_v[i, pl.ds(0, L)]
    s_vec = m_vec
    for c in range(1, NFULL):
        ch = scores_v[i, pl.ds(c * L, L)]
        m_vec = jnp.maximum(m_vec, ch)
        s_vec = s_vec + ch
    ch = scores_v[i, pl.ds(TAIL_OFF, L)]
    m_vec = jnp.maximum(m_vec, jnp.where(tail_valid, ch, -jnp.inf))
    s_vec = s_vec + jnp.where(tail_valid, ch, 0.0)
    m = _splat_max(m_vec)

    for c in range(NFULL):
        sc = scores_v[i, pl.ds(c * L, L)]
        oc = order_v[i, pl.ds(c * L, L)]
        plsc.store_scatter(esort_v, [oc - 1], jnp.exp(sc - m))
    sc = scores_v[i, pl.ds(TAIL_OFF, L)]
    oc = order_v[i, pl.ds(TAIL_OFF, L)]
    plsc.store_scatter(esort_v, [oc - 1], jnp.exp(sc - m), mask=tail_valid)

    ez = jnp.where(tail_valid, esort_v[pl.ds(TAIL_OFF, L)], 0.0)
    s_suf = plsc.cumsum(lax.rev(ez, (0,)))
    acc = jnp.where(lanes < L - TAIL_LO, _vlog(s_suf), 0.0)
    carry = _splat_last(s_suf)
    for c in range(NFULL - 1, -1, -1):
        ec = esort_v[pl.ds(c * L, L)]
        s_suf = plsc.cumsum(lax.rev(ec, (0,))) + carry
        acc = acc + _vlog(s_suf)
        carry = _splat_last(s_suf)

    return acc + (m * (N / L) - s_vec)


def _sc_body(scores_hbm, order_hbm, out_hbm, scores_v, order_v, esort_v, out_v):
    info = plsc.get_sparse_core_info()
    nc = info.num_cores
    wid = lax.axis_index("s") * nc + lax.axis_index("c")
    rpw = B // (nc * info.num_subcores)
    base = wid * rpw
    pltpu.sync_copy(scores_hbm.at[pl.ds(base, rpw)], scores_v)
    pltpu.sync_copy(order_hbm.at[pl.ds(base, rpw)], order_v)

    lanes = lax.iota(jnp.int32, L)

    def body(i, acc):
        return acc + _row_loss(scores_v, order_v, esort_v, i, lanes)

    acc = lax.fori_loop(0, rpw, body, jnp.zeros((L,), jnp.float32))
    tot = plsc.cumsum(acc)
    out_v[...] = jnp.where(lanes == L - 1, tot * (1.0 / (B * N)), 0.0)
    pltpu.sync_copy(out_v, out_hbm.at[wid])


def _make_sc_call():
    info = plsc.get_sparse_core_info()
    nw = info.num_cores * info.num_subcores
    rpw = B // nw
    mesh = plsc.VectorSubcoreMesh(core_axis_name="c", subcore_axis_name="s")
    return pl.kernel(
        _sc_body,
        mesh=mesh,
        compiler_params=pltpu.CompilerParams(needs_layout_passes=False),
        out_type=jax.ShapeDtypeStruct((nw, L), jnp.float32),
        scratch_types=[
            pltpu.VMEM((rpw, N), jnp.float32),
            pltpu.VMEM((rpw, N), jnp.int32),
            pltpu.VMEM((N,), jnp.float32),
            pltpu.VMEM((L,), jnp.float32),
        ],
    )


@jax.jit
def kernel(scores, order):
    partials = _make_sc_call()(scores, order)
    return jnp.sum(partials)


# speedup vs baseline: 89.9523x; 1.1295x over previous
import jax
import jax.numpy as jnp
from jax import lax
from jax.experimental import pallas as pl
from jax.experimental.pallas import tpu as pltpu, tpu_sc as plsc

B, N = 1024, 200
L = 16
NFULL = N // L
TAIL_OFF = N - L
TAIL_LO = L - (N - NFULL * L)
NCHUNK = NFULL + 1

_LN2 = 0.6931471805599453
_SQRT2 = 1.4142135623730951
_MANT = 0x007FFFFF
_ONE_BITS = 0x3F800000


def _vlog(x):
    bits = plsc.bitcast(x, jnp.int32)
    k = lax.shift_right_logical(bits, 23) - 127
    fbits = lax.bitwise_or(lax.bitwise_and(bits, _MANT), _ONE_BITS)
    f = plsc.bitcast(fbits, jnp.float32)
    big = f > _SQRT2
    f = jnp.where(big, f * 0.5, f)
    kf = k.astype(jnp.float32) + jnp.where(big, 1.0, 0.0)
    t = (f - 1.0) / (f + 1.0)
    t2 = t * t
    p = 0.14285715 + t2 * 0.11111111
    p = 0.2 + t2 * p
    p = 0.33333334 + t2 * p
    return kf * _LN2 + (t + t * t2 * p) * 2.0


def _gather(v, idx):
    return v.at[idx].get(mode="promise_in_bounds")


def _splat_max(v, lanes):
    for sh in (1, 2, 4, 8):
        v = jnp.maximum(v, _gather(v, lax.bitwise_xor(lanes, sh)))
    return v


def _row_scatter(scores_v, order_v, ssort_v, i, lanes):
    tail_valid = lanes >= TAIL_LO
    m_vec = scores_v[i, pl.ds(0, L)]
    s_vec = m_vec
    oc = order_v[i, pl.ds(0, L)]
    plsc.store_scatter(ssort_v, [oc - 1], m_vec)
    for c in range(1, NFULL):
        ch = scores_v[i, pl.ds(c * L, L)]
        oc = order_v[i, pl.ds(c * L, L)]
        plsc.store_scatter(ssort_v, [oc - 1], ch)
        m_vec = jnp.maximum(m_vec, ch)
        s_vec = s_vec + ch
    ch = scores_v[i, pl.ds(TAIL_OFF, L)]
    oc = order_v[i, pl.ds(TAIL_OFF, L)]
    plsc.store_scatter(ssort_v, [oc - 1], ch, mask=tail_valid)
    m_vec = jnp.maximum(m_vec, jnp.where(tail_valid, ch, -jnp.inf))
    s_vec = s_vec + jnp.where(tail_valid, ch, 0.0)
    return _splat_max(m_vec, lanes), s_vec


def _row_suffix(ssort_v, m, lanes):
    tail_valid = lanes >= TAIL_LO
    idx_last = jnp.full((L,), L - 1, jnp.int32)

    ch = ssort_v[pl.ds(TAIL_OFF, L)]
    ez = jnp.where(tail_valid, jnp.exp(ch - m), 0.0)
    s_suf = plsc.cumsum(lax.rev(ez, (0,)))
    carry = _gather(s_suf, idx_last)
    s_suf = jnp.where(lanes < L - TAIL_LO, s_suf, 1.0)
    bits = plsc.bitcast(s_suf, jnp.int32)
    eacc = lax.shift_right_logical(bits, 23)
    rp = plsc.bitcast(lax.bitwise_or(lax.bitwise_and(bits, _MANT), _ONE_BITS),
                      jnp.float32)
    for c in range(NFULL - 1, -1, -1):
        ch = ssort_v[pl.ds(c * L, L)]
        e = jnp.exp(ch - m)
        s_suf = plsc.cumsum(lax.rev(e, (0,))) + carry
        carry = _gather(s_suf, idx_last)
        bits = plsc.bitcast(s_suf, jnp.int32)
        eacc = eacc + lax.shift_right_logical(bits, 23)
        rp = rp * plsc.bitcast(
            lax.bitwise_or(lax.bitwise_and(bits, _MANT), _ONE_BITS), jnp.float32)

    kf = (eacc - 127 * NCHUNK).astype(jnp.float32)
    return kf * _LN2 + _vlog(rp) + m * (N / L)


def _sc_body(scores_hbm, order_hbm, out_hbm, scores_v, order_v, ssort0_v,
             ssort1_v, out_v):
    info = plsc.get_sparse_core_info()
    nc = info.num_cores
    wid = lax.axis_index("s") * nc + lax.axis_index("c")
    rpw = B // (nc * info.num_subcores)
    base = wid * rpw
    pltpu.sync_copy(scores_hbm.at[pl.ds(base, rpw)], scores_v)
    pltpu.sync_copy(order_hbm.at[pl.ds(base, rpw)], order_v)

    lanes = lax.iota(jnp.int32, L)

    def body(i, acc):
        m0, s0 = _row_scatter(scores_v, order_v, ssort0_v, 2 * i, lanes)
        m1, s1 = _row_scatter(scores_v, order_v, ssort1_v, 2 * i + 1, lanes)
        acc = acc - s0 - s1
        acc = acc + _row_suffix(ssort0_v, m0, lanes)
        return acc + _row_suffix(ssort1_v, m1, lanes)

    acc = lax.fori_loop(0, rpw // 2, body, jnp.zeros((L,), jnp.float32))
    tot = plsc.cumsum(acc)
    out_v[...] = jnp.where(lanes == L - 1, tot * (1.0 / (B * N)), 0.0)
    pltpu.sync_copy(out_v, out_hbm.at[wid])


def _make_sc_call():
    info = plsc.get_sparse_core_info()
    nw = info.num_cores * info.num_subcores
    rpw = B // nw
    mesh = plsc.VectorSubcoreMesh(core_axis_name="c", subcore_axis_name="s")
    return pl.kernel(
        _sc_body,
        mesh=mesh,
        compiler_params=pltpu.CompilerParams(needs_layout_passes=False),
        out_type=jax.ShapeDtypeStruct((nw, L), jnp.float32),
        scratch_types=[
            pltpu.VMEM((rpw, N), jnp.float32),
            pltpu.VMEM((rpw, N), jnp.int32),
            pltpu.VMEM((N,), jnp.float32),
            pltpu.VMEM((N,), jnp.float32),
            pltpu.VMEM((L,), jnp.float32),
        ],
    )


@jax.jit
def kernel(scores, order):
    partials = _make_sc_call()(scores, order)
    return jnp.sum(partials)
